# baseline (speedup 1.0000x reference)
import jax
import jax.numpy as jnp
from jax.experimental import pallas as pl
from jax.experimental.pallas import tpu as pltpu

B = 256
N = 256
EPG = 512
IN_C = 5
HID = 512
STEPS = 4


def _count_kernel(src_ref, dst_ref, out_ref):
    eq = (src_ref[...] == dst_ref[...]).astype(jnp.float32)
    t = jnp.sum(eq, axis=1, keepdims=True)
    out_ref[...] = jnp.sum(t, axis=0, keepdims=True)


def _main_kernel(cnt_ref, xT_ref, srow_ref, drow_ref, scol_ref, dcol_ref,
                 fdx_ref, nats_ref,
                 linT_ref, linb_ref, g1cat_ref, g1b_ref, g2T_ref, g2b_ref,
                 flT_ref, flb_ref, m1T_ref, m1b_ref, m2T_ref, m2b_ref,
                 m3T_ref, m3b_ref, out_ref):
    p = pl.program_id(0)
    x = xT_ref[0]
    srow = srow_ref[0]
    drow = drow_ref[0]
    scol = scol_ref[0]
    dcol = dcol_ref[0]

    keep_row = srow != drow
    keep_col = scol != dcol
    n_iota_r = jax.lax.broadcasted_iota(jnp.int32, (N, EPG), 0)
    ST = jnp.where((n_iota_r == srow) & keep_row, 1.0, 0.0)
    n_iota_c = jax.lax.broadcasted_iota(jnp.int32, (EPG, N), 1)
    D = jnp.where((n_iota_c == dcol) & keep_col, 1.0, 0.0)

    c_extra = jnp.where(p == 0, cnt_ref[...], 0.0)
    ir = jax.lax.broadcasted_iota(jnp.int32, (N, N), 0)
    ic = jax.lax.broadcasted_iota(jnp.int32, (N, N), 1)
    eye = jnp.where(ir == ic, 1.0, 0.0)
    eye_d = eye + jnp.where((ir == 0) & (ic == 0), c_extra, 0.0)
    ST_full = jnp.concatenate([ST, eye], axis=1)
    D_full = jnp.concatenate([D, eye_d], axis=0)

    def prop(v):
        tmp = jnp.dot(v, ST_full, preferred_element_type=jnp.float32,
                precision=jax.lax.Precision.HIGHEST)
        return jnp.dot(tmp, D_full, preferred_element_type=jnp.float32,
                precision=jax.lax.Precision.HIGHEST)

    for gc in range(STEPS):
        if gc > 0:
            h = jnp.tanh(jnp.dot(linT_ref[gc], x,
                                 preferred_element_type=jnp.float32,
                precision=jax.lax.Precision.HIGHEST)
                         + linb_ref[gc])
            h_hi = h.astype(jnp.bfloat16)
            h_lo = (h - h_hi.astype(jnp.float32)).astype(jnp.bfloat16)
            b3 = jnp.concatenate([h_hi, h_lo, h_hi], axis=0)
            h = jnp.tanh(jnp.dot(g1cat_ref[gc], b3,
                                 preferred_element_type=jnp.float32)
                         + g1b_ref[gc])
            x = jnp.dot(g2T_ref[gc], h,
                        preferred_element_type=jnp.float32,
                precision=jax.lax.Precision.HIGHEST) + g2b_ref[gc]
        x = prop(x)

    lane = jax.lax.broadcasted_iota(jnp.int32, (1, N), 1)
    mask = (lane < nats_ref[0]) & (fdx_ref[0] == 1)
    m = jnp.max(jnp.where(mask, x, -jnp.inf), axis=1, keepdims=True)
    m = jnp.where(jnp.isfinite(m), m, 0.0)

    h = jnp.tanh(jnp.dot(flT_ref[...], m,
                         preferred_element_type=jnp.float32,
                precision=jax.lax.Precision.HIGHEST) + flb_ref[...])
    h = jnp.tanh(jnp.dot(m1T_ref[...], h,
                         preferred_element_type=jnp.float32,
                precision=jax.lax.Precision.HIGHEST) + m1b_ref[...])
    h = jnp.tanh(jnp.dot(m2T_ref[...], h,
                         preferred_element_type=jnp.float32,
                precision=jax.lax.Precision.HIGHEST) + m2b_ref[...])
    o = jnp.dot(m3T_ref[...], h,
                preferred_element_type=jnp.float32,
                precision=jax.lax.Precision.HIGHEST) + m3b_ref[...]
    out_ref[pl.ds(p, 1), :] = o


def _full_spec(shape):
    nd = len(shape)
    return pl.BlockSpec(shape, lambda p, _nd=nd: (0,) * _nd)


def kernel(inputs, labels, rval, findex, nats, lin_W, lin_b, g1_W, g1_b,
           g2_W, g2_b, fl_W, fl_b, m1_W, m1_b, m2_W, m2_b, m3_W, m3_b):
    src = labels[:, :, 0]
    dst = labels[:, :, 1]
    srow = src.reshape(B, 1, EPG)
    drow = dst.reshape(B, 1, EPG)
    scol = src.reshape(B, EPG, 1)
    dcol = dst.reshape(B, EPG, 1)
    xT = inputs.transpose(0, 2, 1)
    fdx = findex[:, :, 0].reshape(B, 1, N)
    natsr = nats.reshape(B, 1, 1)

    linT = lin_W.transpose(0, 2, 1)
    linb = lin_b[:, :, None]
    g1T = g1_W.transpose(0, 2, 1)
    g1T_hi = g1T.astype(jnp.bfloat16)
    g1T_lo = (g1T - g1T_hi.astype(jnp.float32)).astype(jnp.bfloat16)
    g1cat = jnp.concatenate([g1T_hi, g1T_hi, g1T_lo], axis=2)
    g1b = g1_b[:, :, None]
    g2T = g2_W.transpose(0, 2, 1)
    g2b = g2_b[:, :, None]
    flT = fl_W.T
    flb = fl_b[:, None]
    m1T = m1_W.T
    m1b = m1_b[:, None]
    m2T = m2_W.T
    m2b = m2_b[:, None]
    m3T = m3_W.T
    m3b = m3_b[:, None]

    cnt = pl.pallas_call(
        _count_kernel,
        out_shape=jax.ShapeDtypeStruct((1, 1), jnp.float32),
    )(src, dst)

    grid = (B,)
    in_specs = [
        _full_spec((1, 1)),
        pl.BlockSpec((1, IN_C, N), lambda p: (p, 0, 0)),
        pl.BlockSpec((1, 1, EPG), lambda p: (p, 0, 0)),
        pl.BlockSpec((1, 1, EPG), lambda p: (p, 0, 0)),
        pl.BlockSpec((1, EPG, 1), lambda p: (p, 0, 0)),
        pl.BlockSpec((1, EPG, 1), lambda p: (p, 0, 0)),
        pl.BlockSpec((1, 1, N), lambda p: (p, 0, 0)),
        pl.BlockSpec((1, 1, 1), lambda p: (p, 0, 0)),
        _full_spec((STEPS, HID, IN_C)),
        _full_spec((STEPS, HID, 1)),
        _full_spec((STEPS, HID, 3 * HID)),
        _full_spec((STEPS, HID, 1)),
        _full_spec((STEPS, IN_C, HID)),
        _full_spec((STEPS, IN_C, 1)),
        _full_spec((64, IN_C)),
        _full_spec((64, 1)),
        _full_spec((32, 64)),
        _full_spec((32, 1)),
        _full_spec((16, 32)),
        _full_spec((16, 1)),
        _full_spec((1, 16)),
        _full_spec((1, 1)),
    ]
    out = pl.pallas_call(
        _main_kernel,
        grid=grid,
        in_specs=in_specs,
        out_specs=pl.BlockSpec((B, 1), lambda p: (0, 0)),
        out_shape=jax.ShapeDtypeStruct((B, 1), jnp.float32),
        compiler_params=pltpu.CompilerParams(
            dimension_semantics=("arbitrary",),
        ),
    )(cnt, xT, srow, drow, scol, dcol, fdx, natsr,
      linT, linb, g1cat, g1b, g2T, g2b,
      flT, flb, m1T, m1b, m2T, m2b, m3T, m3b)
    return out


# speedup vs baseline: 1.2417x; 1.2417x over previous
import jax
import jax.numpy as jnp
from jax.experimental import pallas as pl
from jax.experimental.pallas import tpu as pltpu

B = 256
N = 256
EPG = 512
IN_C = 5
HID = 512
STEPS = 4


def _count_kernel(src_ref, dst_ref, out_ref):
    eq = (src_ref[...] == dst_ref[...]).astype(jnp.float32)
    t = jnp.sum(eq, axis=1, keepdims=True)
    out_ref[...] = jnp.sum(t, axis=0, keepdims=True)


def _main_kernel(cnt_ref, xT_ref, srow_ref, drow_ref, scol_ref, dcol_ref,
                 fdx_ref, nats_ref,
                 lincat_ref, linb_ref, g1cat_ref, g1b_ref, g2cat_ref, g2b_ref,
                 flT_ref, flb_ref, m1T_ref, m1b_ref, m2T_ref, m2b_ref,
                 m3T_ref, m3b_ref, out_ref):
    p = pl.program_id(0)
    x = xT_ref[0]
    srow = srow_ref[0]
    drow = drow_ref[0]
    scol = scol_ref[0]
    dcol = dcol_ref[0]

    keep_row = srow != drow
    keep_col = scol != dcol
    n_iota_r = jax.lax.broadcasted_iota(jnp.int32, (N, EPG), 0)
    ST = jnp.where((n_iota_r == srow) & keep_row, 1.0, 0.0)
    n_iota_c = jax.lax.broadcasted_iota(jnp.int32, (EPG, N), 1)
    D = jnp.where((n_iota_c == dcol) & keep_col, 1.0, 0.0)

    c_extra = jnp.where(p == 0, cnt_ref[...], 0.0)
    ir = jax.lax.broadcasted_iota(jnp.int32, (N, N), 0)
    ic = jax.lax.broadcasted_iota(jnp.int32, (N, N), 1)
    eye = jnp.where(ir == ic, 1.0, 0.0)
    eye_d = eye + jnp.where((ir == 0) & (ic == 0), c_extra, 0.0)
    ST_full = jnp.concatenate([ST, eye], axis=1)
    D_full = jnp.concatenate([D, eye_d], axis=0)

    def prop(v):
        tmp = jnp.dot(v, ST_full, preferred_element_type=jnp.float32,
                precision=jax.lax.Precision.HIGHEST)
        return jnp.dot(tmp, D_full, preferred_element_type=jnp.float32,
                precision=jax.lax.Precision.HIGHEST)

    for gc in range(STEPS):
        if gc > 0:
            x_hi = x.astype(jnp.bfloat16)
            x_lo = (x - x_hi.astype(jnp.float32)).astype(jnp.bfloat16)
            xb3 = jnp.concatenate([x_hi, x_lo, x_hi], axis=0)
            h = jnp.tanh(jnp.dot(lincat_ref[gc], xb3,
                                 preferred_element_type=jnp.float32)
                         + linb_ref[gc])
            h_hi = h.astype(jnp.bfloat16)
            h_lo = (h - h_hi.astype(jnp.float32)).astype(jnp.bfloat16)
            b3 = jnp.concatenate([h_hi, h_lo, h_hi], axis=0)
            h = jnp.tanh(jnp.dot(g1cat_ref[gc], b3,
                                 preferred_element_type=jnp.float32)
                         + g1b_ref[gc])
            h2_hi = h.astype(jnp.bfloat16)
            h2_lo = (h - h2_hi.astype(jnp.float32)).astype(jnp.bfloat16)
            hb3 = jnp.concatenate([h2_hi, h2_lo, h2_hi], axis=0)
            x = jnp.dot(g2cat_ref[gc], hb3,
                        preferred_element_type=jnp.float32) + g2b_ref[gc]
        x = prop(x)

    lane = jax.lax.broadcasted_iota(jnp.int32, (1, N), 1)
    mask = (lane < nats_ref[0]) & (fdx_ref[0] == 1)
    m = jnp.max(jnp.where(mask, x, -jnp.inf), axis=1, keepdims=True)
    m = jnp.where(jnp.isfinite(m), m, 0.0)

    h = jnp.tanh(jnp.dot(flT_ref[...], m,
                         preferred_element_type=jnp.float32,
                precision=jax.lax.Precision.HIGHEST) + flb_ref[...])
    h = jnp.tanh(jnp.dot(m1T_ref[...], h,
                         preferred_element_type=jnp.float32,
                precision=jax.lax.Precision.HIGHEST) + m1b_ref[...])
    h = jnp.tanh(jnp.dot(m2T_ref[...], h,
                         preferred_element_type=jnp.float32,
                precision=jax.lax.Precision.HIGHEST) + m2b_ref[...])
    o = jnp.dot(m3T_ref[...], h,
                preferred_element_type=jnp.float32,
                precision=jax.lax.Precision.HIGHEST) + m3b_ref[...]
    out_ref[pl.ds(p, 1), :] = o


def _full_spec(shape):
    nd = len(shape)
    return pl.BlockSpec(shape, lambda p, _nd=nd: (0,) * _nd)


def kernel(inputs, labels, rval, findex, nats, lin_W, lin_b, g1_W, g1_b,
           g2_W, g2_b, fl_W, fl_b, m1_W, m1_b, m2_W, m2_b, m3_W, m3_b):
    src = labels[:, :, 0]
    dst = labels[:, :, 1]
    srow = src.reshape(B, 1, EPG)
    drow = dst.reshape(B, 1, EPG)
    scol = src.reshape(B, EPG, 1)
    dcol = dst.reshape(B, EPG, 1)
    xT = inputs.transpose(0, 2, 1)
    fdx = findex[:, :, 0].reshape(B, 1, N)
    natsr = nats.reshape(B, 1, 1)

    linT = lin_W.transpose(0, 2, 1)
    linT_hi = linT.astype(jnp.bfloat16)
    linT_lo = (linT - linT_hi.astype(jnp.float32)).astype(jnp.bfloat16)
    lincat = jnp.concatenate([linT_hi, linT_hi, linT_lo], axis=2)
    linb = lin_b[:, :, None]
    g1T = g1_W.transpose(0, 2, 1)
    g1T_hi = g1T.astype(jnp.bfloat16)
    g1T_lo = (g1T - g1T_hi.astype(jnp.float32)).astype(jnp.bfloat16)
    g1cat = jnp.concatenate([g1T_hi, g1T_hi, g1T_lo], axis=2)
    g1b = g1_b[:, :, None]
    g2T = g2_W.transpose(0, 2, 1)
    g2T_hi = g2T.astype(jnp.bfloat16)
    g2T_lo = (g2T - g2T_hi.astype(jnp.float32)).astype(jnp.bfloat16)
    g2cat = jnp.concatenate([g2T_hi, g2T_hi, g2T_lo], axis=2)
    g2b = g2_b[:, :, None]
    flT = fl_W.T
    flb = fl_b[:, None]
    m1T = m1_W.T
    m1b = m1_b[:, None]
    m2T = m2_W.T
    m2b = m2_b[:, None]
    m3T = m3_W.T
    m3b = m3_b[:, None]

    cnt = pl.pallas_call(
        _count_kernel,
        out_shape=jax.ShapeDtypeStruct((1, 1), jnp.float32),
    )(src, dst)

    grid = (B,)
    in_specs = [
        _full_spec((1, 1)),
        pl.BlockSpec((1, IN_C, N), lambda p: (p, 0, 0)),
        pl.BlockSpec((1, 1, EPG), lambda p: (p, 0, 0)),
        pl.BlockSpec((1, 1, EPG), lambda p: (p, 0, 0)),
        pl.BlockSpec((1, EPG, 1), lambda p: (p, 0, 0)),
        pl.BlockSpec((1, EPG, 1), lambda p: (p, 0, 0)),
        pl.BlockSpec((1, 1, N), lambda p: (p, 0, 0)),
        pl.BlockSpec((1, 1, 1), lambda p: (p, 0, 0)),
        _full_spec((STEPS, HID, 3 * IN_C)),
        _full_spec((STEPS, HID, 1)),
        _full_spec((STEPS, HID, 3 * HID)),
        _full_spec((STEPS, HID, 1)),
        _full_spec((STEPS, IN_C, 3 * HID)),
        _full_spec((STEPS, IN_C, 1)),
        _full_spec((64, IN_C)),
        _full_spec((64, 1)),
        _full_spec((32, 64)),
        _full_spec((32, 1)),
        _full_spec((16, 32)),
        _full_spec((16, 1)),
        _full_spec((1, 16)),
        _full_spec((1, 1)),
    ]
    out = pl.pallas_call(
        _main_kernel,
        grid=grid,
        in_specs=in_specs,
        out_specs=pl.BlockSpec((B, 1), lambda p: (0, 0)),
        out_shape=jax.ShapeDtypeStruct((B, 1), jnp.float32),
        compiler_params=pltpu.CompilerParams(
            dimension_semantics=("arbitrary",),
        ),
    )(cnt, xT, srow, drow, scol, dcol, fdx, natsr,
      lincat, linb, g1cat, g1b, g2cat, g2b,
      flT, flb, m1T, m1b, m2T, m2b, m3T, m3b)
    return out


# speedup vs baseline: 2.2877x; 1.8424x over previous
import jax
import jax.numpy as jnp
from jax.experimental import pallas as pl
from jax.experimental.pallas import tpu as pltpu

B = 256
N = 256
EPG = 512
IN_C = 5
HID = 512
STEPS = 4
G = 4


def _count_kernel(src_ref, dst_ref, out_ref):
    eq = (src_ref[...] == dst_ref[...]).astype(jnp.float32)
    t = jnp.sum(eq, axis=1, keepdims=True)
    out_ref[...] = jnp.sum(t, axis=0, keepdims=True)


def _main_kernel(cnt_ref, xT_ref, srow_ref, drow_ref, scol_ref, dcol_ref,
                 fdx_ref, nats_ref,
                 lincat_ref, linb_ref, g1cat_ref, g1b_ref, g2cat_ref, g2b_ref,
                 flT_ref, flb_ref, m1T_ref, m1b_ref, m2T_ref, m2b_ref,
                 m3T_ref, m3b_ref, out_ref):
    p = pl.program_id(0)
    x = jnp.concatenate([xT_ref[g] for g in range(G)], axis=1)

    n_iota_r = jax.lax.broadcasted_iota(jnp.int32, (N, EPG), 0)
    n_iota_c = jax.lax.broadcasted_iota(jnp.int32, (EPG, N), 1)
    ir = jax.lax.broadcasted_iota(jnp.int32, (N, N), 0)
    ic = jax.lax.broadcasted_iota(jnp.int32, (N, N), 1)
    eye = jnp.where(ir == ic, 1.0, 0.0)
    STs, Ds = [], []
    for g in range(G):
        srow = srow_ref[g]
        drow = drow_ref[g]
        scol = scol_ref[g]
        dcol = dcol_ref[g]
        ST = jnp.where((n_iota_r == srow) & (srow != drow), 1.0, 0.0)
        D = jnp.where((n_iota_c == dcol) & (scol != dcol), 1.0, 0.0)
        c_extra = jnp.where((p == 0) & (g == 0), cnt_ref[...], 0.0)
        eye_d = eye + jnp.where((ir == 0) & (ic == 0), c_extra, 0.0)
        STs.append(jnp.concatenate([ST, eye], axis=1))
        Ds.append(jnp.concatenate([D, eye_d], axis=0))

    def prop(v):
        outs = []
        for g in range(G):
            vg = v[:, g * N:(g + 1) * N]
            tmp = jnp.dot(vg, STs[g], preferred_element_type=jnp.float32,
                    precision=jax.lax.Precision.HIGHEST)
            outs.append(jnp.dot(tmp, Ds[g],
                    preferred_element_type=jnp.float32,
                    precision=jax.lax.Precision.HIGHEST))
        return jnp.concatenate(outs, axis=1)

    for gc in range(STEPS):
        if gc > 0:
            x_hi = x.astype(jnp.bfloat16)
            x_lo = (x - x_hi.astype(jnp.float32)).astype(jnp.bfloat16)
            xb3 = jnp.concatenate([x_hi, x_lo, x_hi], axis=0)
            h = jnp.tanh(jnp.dot(lincat_ref[gc], xb3,
                                 preferred_element_type=jnp.float32)
                         + linb_ref[gc])
            h_hi = h.astype(jnp.bfloat16)
            h_lo = (h - h_hi.astype(jnp.float32)).astype(jnp.bfloat16)
            b3 = jnp.concatenate([h_hi, h_lo, h_hi], axis=0)
            h = jnp.tanh(jnp.dot(g1cat_ref[gc], b3,
                                 preferred_element_type=jnp.float32)
                         + g1b_ref[gc])
            h2_hi = h.astype(jnp.bfloat16)
            h2_lo = (h - h2_hi.astype(jnp.float32)).astype(jnp.bfloat16)
            hb3 = jnp.concatenate([h2_hi, h2_lo, h2_hi], axis=0)
            x = jnp.dot(g2cat_ref[gc], hb3,
                        preferred_element_type=jnp.float32) + g2b_ref[gc]
        x = prop(x)

    lane = jax.lax.broadcasted_iota(jnp.int32, (1, N), 1)
    ms = []
    for g in range(G):
        xg = x[:, g * N:(g + 1) * N]
        mask = (lane < nats_ref[g]) & (fdx_ref[g] == 1)
        m = jnp.max(jnp.where(mask, xg, -jnp.inf), axis=1, keepdims=True)
        ms.append(jnp.where(jnp.isfinite(m), m, 0.0))
    m = jnp.concatenate(ms, axis=1)

    h = jnp.tanh(jnp.dot(flT_ref[...], m,
                         preferred_element_type=jnp.float32,
                precision=jax.lax.Precision.HIGHEST) + flb_ref[...])
    h = jnp.tanh(jnp.dot(m1T_ref[...], h,
                         preferred_element_type=jnp.float32,
                precision=jax.lax.Precision.HIGHEST) + m1b_ref[...])
    h = jnp.tanh(jnp.dot(m2T_ref[...], h,
                         preferred_element_type=jnp.float32,
                precision=jax.lax.Precision.HIGHEST) + m2b_ref[...])
    o = jnp.dot(m3T_ref[...], h,
                preferred_element_type=jnp.float32,
                precision=jax.lax.Precision.HIGHEST) + m3b_ref[...]
    out_ref[0] = o


def _full_spec(shape):
    nd = len(shape)
    return pl.BlockSpec(shape, lambda p, _nd=nd: (0,) * _nd)


def kernel(inputs, labels, rval, findex, nats, lin_W, lin_b, g1_W, g1_b,
           g2_W, g2_b, fl_W, fl_b, m1_W, m1_b, m2_W, m2_b, m3_W, m3_b):
    src = labels[:, :, 0]
    dst = labels[:, :, 1]
    srow = src.reshape(B, 1, EPG)
    drow = dst.reshape(B, 1, EPG)
    scol = src.reshape(B, EPG, 1)
    dcol = dst.reshape(B, EPG, 1)
    xT = inputs.transpose(0, 2, 1)
    fdx = findex[:, :, 0].reshape(B, 1, N)
    natsr = nats.reshape(B, 1, 1)

    linT = lin_W.transpose(0, 2, 1)
    linT_hi = linT.astype(jnp.bfloat16)
    linT_lo = (linT - linT_hi.astype(jnp.float32)).astype(jnp.bfloat16)
    lincat = jnp.concatenate([linT_hi, linT_hi, linT_lo], axis=2)
    linb = lin_b[:, :, None]
    g1T = g1_W.transpose(0, 2, 1)
    g1T_hi = g1T.astype(jnp.bfloat16)
    g1T_lo = (g1T - g1T_hi.astype(jnp.float32)).astype(jnp.bfloat16)
    g1cat = jnp.concatenate([g1T_hi, g1T_hi, g1T_lo], axis=2)
    g1b = g1_b[:, :, None]
    g2T = g2_W.transpose(0, 2, 1)
    g2T_hi = g2T.astype(jnp.bfloat16)
    g2T_lo = (g2T - g2T_hi.astype(jnp.float32)).astype(jnp.bfloat16)
    g2cat = jnp.concatenate([g2T_hi, g2T_hi, g2T_lo], axis=2)
    g2b = g2_b[:, :, None]
    flT = fl_W.T
    flb = fl_b[:, None]
    m1T = m1_W.T
    m1b = m1_b[:, None]
    m2T = m2_W.T
    m2b = m2_b[:, None]
    m3T = m3_W.T
    m3b = m3_b[:, None]

    cnt = pl.pallas_call(
        _count_kernel,
        out_shape=jax.ShapeDtypeStruct((1, 1), jnp.float32),
    )(src, dst)

    grid = (B // G,)
    in_specs = [
        _full_spec((1, 1)),
        pl.BlockSpec((G, IN_C, N), lambda p: (p, 0, 0)),
        pl.BlockSpec((G, 1, EPG), lambda p: (p, 0, 0)),
        pl.BlockSpec((G, 1, EPG), lambda p: (p, 0, 0)),
        pl.BlockSpec((G, EPG, 1), lambda p: (p, 0, 0)),
        pl.BlockSpec((G, EPG, 1), lambda p: (p, 0, 0)),
        pl.BlockSpec((G, 1, N), lambda p: (p, 0, 0)),
        pl.BlockSpec((G, 1, 1), lambda p: (p, 0, 0)),
        _full_spec((STEPS, HID, 3 * IN_C)),
        _full_spec((STEPS, HID, 1)),
        _full_spec((STEPS, HID, 3 * HID)),
        _full_spec((STEPS, HID, 1)),
        _full_spec((STEPS, IN_C, 3 * HID)),
        _full_spec((STEPS, IN_C, 1)),
        _full_spec((64, IN_C)),
        _full_spec((64, 1)),
        _full_spec((32, 64)),
        _full_spec((32, 1)),
        _full_spec((16, 32)),
        _full_spec((16, 1)),
        _full_spec((1, 16)),
        _full_spec((1, 1)),
    ]
    out = pl.pallas_call(
        _main_kernel,
        grid=grid,
        in_specs=in_specs,
        out_specs=pl.BlockSpec((1, 1, G), lambda p: (p, 0, 0)),
        out_shape=jax.ShapeDtypeStruct((B // G, 1, G), jnp.float32),
        compiler_params=pltpu.CompilerParams(
            dimension_semantics=("arbitrary",),
        ),
    )(cnt, xT, srow, drow, scol, dcol, fdx, natsr,
      lincat, linb, g1cat, g1b, g2cat, g2b,
      flT, flb, m1T, m1b, m2T, m2b, m3T, m3b)
    return out.reshape(B, 1)


# speedup vs baseline: 2.3453x; 1.0252x over previous
import jax
import jax.numpy as jnp
from jax.experimental import pallas as pl
from jax.experimental.pallas import tpu as pltpu

B = 256
N = 256
EPG = 512
IN_C = 5
HID = 512
STEPS = 4
G = 8


def _count_kernel(src_ref, dst_ref, out_ref):
    eq = (src_ref[...] == dst_ref[...]).astype(jnp.float32)
    t = jnp.sum(eq, axis=1, keepdims=True)
    out_ref[...] = jnp.sum(t, axis=0, keepdims=True)


def _main_kernel(cnt_ref, xT_ref, srow_ref, drow_ref, scol_ref, dcol_ref,
                 fdx_ref, nats_ref,
                 lincat_ref, linb_ref, g1cat_ref, g1b_ref, g2cat_ref, g2b_ref,
                 flT_ref, flb_ref, m1T_ref, m1b_ref, m2T_ref, m2b_ref,
                 m3T_ref, m3b_ref, out_ref):
    p = pl.program_id(0)
    x = jnp.concatenate([xT_ref[g] for g in range(G)], axis=1)

    n_iota_r = jax.lax.broadcasted_iota(jnp.int32, (N, EPG), 0)
    n_iota_c = jax.lax.broadcasted_iota(jnp.int32, (EPG, N), 1)
    ir = jax.lax.broadcasted_iota(jnp.int32, (N, N), 0)
    ic = jax.lax.broadcasted_iota(jnp.int32, (N, N), 1)
    eye = jnp.where(ir == ic, 1.0, 0.0)
    STs, Ds = [], []
    for g in range(G):
        srow = srow_ref[g]
        drow = drow_ref[g]
        scol = scol_ref[g]
        dcol = dcol_ref[g]
        ST = jnp.where((n_iota_r == srow) & (srow != drow), 1.0, 0.0)
        D = jnp.where((n_iota_c == dcol) & (scol != dcol), 1.0, 0.0)
        c_extra = jnp.where((p == 0) & (g == 0), cnt_ref[...], 0.0)
        eye_d = eye + jnp.where((ir == 0) & (ic == 0), c_extra, 0.0)
        STs.append(jnp.concatenate([ST, eye], axis=1))
        Ds.append(jnp.concatenate([D, eye_d], axis=0))

    def prop(v):
        outs = []
        for g in range(G):
            vg = v[:, g * N:(g + 1) * N]
            tmp = jnp.dot(vg, STs[g], preferred_element_type=jnp.float32,
                    precision=jax.lax.Precision.HIGHEST)
            outs.append(jnp.dot(tmp, Ds[g],
                    preferred_element_type=jnp.float32,
                    precision=jax.lax.Precision.HIGHEST))
        return jnp.concatenate(outs, axis=1)

    for gc in range(STEPS):
        if gc > 0:
            x_hi = x.astype(jnp.bfloat16)
            x_lo = (x - x_hi.astype(jnp.float32)).astype(jnp.bfloat16)
            xb3 = jnp.concatenate([x_hi, x_lo, x_hi], axis=0)
            h = jnp.tanh(jnp.dot(lincat_ref[gc], xb3,
                                 preferred_element_type=jnp.float32)
                         + linb_ref[gc])
            h_hi = h.astype(jnp.bfloat16)
            h_lo = (h - h_hi.astype(jnp.float32)).astype(jnp.bfloat16)
            b3 = jnp.concatenate([h_hi, h_lo, h_hi], axis=0)
            h = jnp.tanh(jnp.dot(g1cat_ref[gc], b3,
                                 preferred_element_type=jnp.float32)
                         + g1b_ref[gc])
            h2_hi = h.astype(jnp.bfloat16)
            h2_lo = (h - h2_hi.astype(jnp.float32)).astype(jnp.bfloat16)
            hb3 = jnp.concatenate([h2_hi, h2_lo, h2_hi], axis=0)
            x = jnp.dot(g2cat_ref[gc], hb3,
                        preferred_element_type=jnp.float32) + g2b_ref[gc]
        x = prop(x)

    lane = jax.lax.broadcasted_iota(jnp.int32, (1, N), 1)
    ms = []
    for g in range(G):
        xg = x[:, g * N:(g + 1) * N]
        mask = (lane < nats_ref[g]) & (fdx_ref[g] == 1)
        m = jnp.max(jnp.where(mask, xg, -jnp.inf), axis=1, keepdims=True)
        ms.append(jnp.where(jnp.isfinite(m), m, 0.0))
    m = jnp.concatenate(ms, axis=1)

    h = jnp.tanh(jnp.dot(flT_ref[...], m,
                         preferred_element_type=jnp.float32,
                precision=jax.lax.Precision.HIGHEST) + flb_ref[...])
    h = jnp.tanh(jnp.dot(m1T_ref[...], h,
                         preferred_element_type=jnp.float32,
                precision=jax.lax.Precision.HIGHEST) + m1b_ref[...])
    h = jnp.tanh(jnp.dot(m2T_ref[...], h,
                         preferred_element_type=jnp.float32,
                precision=jax.lax.Precision.HIGHEST) + m2b_ref[...])
    o = jnp.dot(m3T_ref[...], h,
                preferred_element_type=jnp.float32,
                precision=jax.lax.Precision.HIGHEST) + m3b_ref[...]
    out_ref[0] = o


def _full_spec(shape):
    nd = len(shape)
    return pl.BlockSpec(shape, lambda p, _nd=nd: (0,) * _nd)


def kernel(inputs, labels, rval, findex, nats, lin_W, lin_b, g1_W, g1_b,
           g2_W, g2_b, fl_W, fl_b, m1_W, m1_b, m2_W, m2_b, m3_W, m3_b):
    src = labels[:, :, 0]
    dst = labels[:, :, 1]
    srow = src.reshape(B, 1, EPG)
    drow = dst.reshape(B, 1, EPG)
    scol = src.reshape(B, EPG, 1)
    dcol = dst.reshape(B, EPG, 1)
    xT = inputs.transpose(0, 2, 1)
    fdx = findex[:, :, 0].reshape(B, 1, N)
    natsr = nats.reshape(B, 1, 1)

    linT = lin_W.transpose(0, 2, 1)
    linT_hi = linT.astype(jnp.bfloat16)
    linT_lo = (linT - linT_hi.astype(jnp.float32)).astype(jnp.bfloat16)
    lincat = jnp.concatenate([linT_hi, linT_hi, linT_lo], axis=2)
    linb = lin_b[:, :, None]
    g1T = g1_W.transpose(0, 2, 1)
    g1T_hi = g1T.astype(jnp.bfloat16)
    g1T_lo = (g1T - g1T_hi.astype(jnp.float32)).astype(jnp.bfloat16)
    g1cat = jnp.concatenate([g1T_hi, g1T_hi, g1T_lo], axis=2)
    g1b = g1_b[:, :, None]
    g2T = g2_W.transpose(0, 2, 1)
    g2T_hi = g2T.astype(jnp.bfloat16)
    g2T_lo = (g2T - g2T_hi.astype(jnp.float32)).astype(jnp.bfloat16)
    g2cat = jnp.concatenate([g2T_hi, g2T_hi, g2T_lo], axis=2)
    g2b = g2_b[:, :, None]
    flT = fl_W.T
    flb = fl_b[:, None]
    m1T = m1_W.T
    m1b = m1_b[:, None]
    m2T = m2_W.T
    m2b = m2_b[:, None]
    m3T = m3_W.T
    m3b = m3_b[:, None]

    cnt = pl.pallas_call(
        _count_kernel,
        out_shape=jax.ShapeDtypeStruct((1, 1), jnp.float32),
    )(src, dst)

    grid = (B // G,)
    in_specs = [
        _full_spec((1, 1)),
        pl.BlockSpec((G, IN_C, N), lambda p: (p, 0, 0)),
        pl.BlockSpec((G, 1, EPG), lambda p: (p, 0, 0)),
        pl.BlockSpec((G, 1, EPG), lambda p: (p, 0, 0)),
        pl.BlockSpec((G, EPG, 1), lambda p: (p, 0, 0)),
        pl.BlockSpec((G, EPG, 1), lambda p: (p, 0, 0)),
        pl.BlockSpec((G, 1, N), lambda p: (p, 0, 0)),
        pl.BlockSpec((G, 1, 1), lambda p: (p, 0, 0)),
        _full_spec((STEPS, HID, 3 * IN_C)),
        _full_spec((STEPS, HID, 1)),
        _full_spec((STEPS, HID, 3 * HID)),
        _full_spec((STEPS, HID, 1)),
        _full_spec((STEPS, IN_C, 3 * HID)),
        _full_spec((STEPS, IN_C, 1)),
        _full_spec((64, IN_C)),
        _full_spec((64, 1)),
        _full_spec((32, 64)),
        _full_spec((32, 1)),
        _full_spec((16, 32)),
        _full_spec((16, 1)),
        _full_spec((1, 16)),
        _full_spec((1, 1)),
    ]
    out = pl.pallas_call(
        _main_kernel,
        grid=grid,
        in_specs=in_specs,
        out_specs=pl.BlockSpec((1, 1, G), lambda p: (p, 0, 0)),
        out_shape=jax.ShapeDtypeStruct((B // G, 1, G), jnp.float32),
        compiler_params=pltpu.CompilerParams(
            dimension_semantics=("arbitrary",),
        ),
    )(cnt, xT, srow, drow, scol, dcol, fdx, natsr,
      lincat, linb, g1cat, g1b, g2cat, g2b,
      flT, flb, m1T, m1b, m2T, m2b, m3T, m3b)
    return out.reshape(B, 1)


# speedup vs baseline: 2.8987x; 1.2360x over previous
import jax
import jax.numpy as jnp
from jax.experimental import pallas as pl
from jax.experimental.pallas import tpu as pltpu

B = 256
N = 256
EPG = 512
IN_C = 5
HID = 512
STEPS = 4
G = 4


def _count_kernel(src_ref, dst_ref, out_ref):
    eq = (src_ref[...] == dst_ref[...]).astype(jnp.float32)
    t = jnp.sum(eq, axis=1, keepdims=True)
    out_ref[...] = jnp.sum(t, axis=0, keepdims=True)


def _main_kernel(cnt_ref, xT_ref, srow_ref, drow_ref, scol_ref, dcol_ref,
                 fdx_ref, nats_ref,
                 lin_bf_ref, linb_ref, g1_bf_ref, g1b_ref, g2_bf_ref, g2b_ref,
                 flT_ref, flb_ref, m1T_ref, m1b_ref, m2T_ref, m2b_ref,
                 m3T_ref, m3b_ref, out_ref):
    p = pl.program_id(0)
    x = jnp.concatenate([xT_ref[g] for g in range(G)], axis=1)

    n_iota_r = jax.lax.broadcasted_iota(jnp.int32, (N, EPG), 0)
    n_iota_c = jax.lax.broadcasted_iota(jnp.int32, (EPG, N), 1)
    ir = jax.lax.broadcasted_iota(jnp.int32, (N, N), 0)
    ic = jax.lax.broadcasted_iota(jnp.int32, (N, N), 1)
    eye = jnp.where(ir == ic, 1.0, 0.0)
    STs, Ds = [], []
    for g in range(G):
        srow = srow_ref[g]
        drow = drow_ref[g]
        scol = scol_ref[g]
        dcol = dcol_ref[g]
        ST = jnp.where((n_iota_r == srow) & (srow != drow), 1.0, 0.0)
        D = jnp.where((n_iota_c == dcol) & (scol != dcol), 1.0, 0.0)
        c_extra = jnp.where((p == 0) & (g == 0), cnt_ref[...], 0.0)
        eye_d = eye + jnp.where((ir == 0) & (ic == 0), c_extra, 0.0)
        STs.append(jnp.concatenate([ST, eye], axis=1))
        Ds.append(jnp.concatenate([D, eye_d], axis=0))

    def prop(v):
        outs = []
        for g in range(G):
            vg = v[:, g * N:(g + 1) * N]
            tmp = jnp.dot(vg, STs[g], preferred_element_type=jnp.float32,
                    precision=jax.lax.Precision.HIGHEST)
            outs.append(jnp.dot(tmp, Ds[g],
                    preferred_element_type=jnp.float32,
                    precision=jax.lax.Precision.HIGHEST))
        return jnp.concatenate(outs, axis=1)

    for gc in range(STEPS):
        if gc > 0:
            h = jnp.tanh(jnp.dot(lin_bf_ref[gc], x.astype(jnp.bfloat16),
                                 preferred_element_type=jnp.float32)
                         + linb_ref[gc])
            h = jnp.tanh(jnp.dot(g1_bf_ref[gc], h.astype(jnp.bfloat16),
                                 preferred_element_type=jnp.float32)
                         + g1b_ref[gc])
            x = jnp.dot(g2_bf_ref[gc], h.astype(jnp.bfloat16),
                        preferred_element_type=jnp.float32) + g2b_ref[gc]
        x = prop(x)

    lane = jax.lax.broadcasted_iota(jnp.int32, (1, N), 1)
    ms = []
    for g in range(G):
        xg = x[:, g * N:(g + 1) * N]
        mask = (lane < nats_ref[g]) & (fdx_ref[g] == 1)
        m = jnp.max(jnp.where(mask, xg, -jnp.inf), axis=1, keepdims=True)
        ms.append(jnp.where(jnp.isfinite(m), m, 0.0))
    m = jnp.concatenate(ms, axis=1)

    h = jnp.tanh(jnp.dot(flT_ref[...], m.astype(jnp.bfloat16),
                         preferred_element_type=jnp.float32) + flb_ref[...])
    h = jnp.tanh(jnp.dot(m1T_ref[...], h.astype(jnp.bfloat16),
                         preferred_element_type=jnp.float32) + m1b_ref[...])
    h = jnp.tanh(jnp.dot(m2T_ref[...], h.astype(jnp.bfloat16),
                         preferred_element_type=jnp.float32) + m2b_ref[...])
    o = jnp.dot(m3T_ref[...], h.astype(jnp.bfloat16),
                preferred_element_type=jnp.float32) + m3b_ref[...]
    out_ref[0] = o


def _full_spec(shape):
    nd = len(shape)
    return pl.BlockSpec(shape, lambda p, _nd=nd: (0,) * _nd)


def kernel(inputs, labels, rval, findex, nats, lin_W, lin_b, g1_W, g1_b,
           g2_W, g2_b, fl_W, fl_b, m1_W, m1_b, m2_W, m2_b, m3_W, m3_b):
    src = labels[:, :, 0]
    dst = labels[:, :, 1]
    srow = src.reshape(B, 1, EPG)
    drow = dst.reshape(B, 1, EPG)
    scol = src.reshape(B, EPG, 1)
    dcol = dst.reshape(B, EPG, 1)
    xT = inputs.transpose(0, 2, 1)
    fdx = findex[:, :, 0].reshape(B, 1, N)
    natsr = nats.reshape(B, 1, 1)

    lin_bf = lin_W.transpose(0, 2, 1).astype(jnp.bfloat16)
    linb = lin_b[:, :, None]
    g1_bf = g1_W.transpose(0, 2, 1).astype(jnp.bfloat16)
    g1b = g1_b[:, :, None]
    g2_bf = g2_W.transpose(0, 2, 1).astype(jnp.bfloat16)
    g2b = g2_b[:, :, None]
    flT = fl_W.T.astype(jnp.bfloat16)
    flb = fl_b[:, None]
    m1T = m1_W.T.astype(jnp.bfloat16)
    m1b = m1_b[:, None]
    m2T = m2_W.T.astype(jnp.bfloat16)
    m2b = m2_b[:, None]
    m3T = m3_W.T.astype(jnp.bfloat16)
    m3b = m3_b[:, None]

    cnt = pl.pallas_call(
        _count_kernel,
        out_shape=jax.ShapeDtypeStruct((1, 1), jnp.float32),
    )(src, dst)

    grid = (B // G,)
    in_specs = [
        _full_spec((1, 1)),
        pl.BlockSpec((G, IN_C, N), lambda p: (p, 0, 0)),
        pl.BlockSpec((G, 1, EPG), lambda p: (p, 0, 0)),
        pl.BlockSpec((G, 1, EPG), lambda p: (p, 0, 0)),
        pl.BlockSpec((G, EPG, 1), lambda p: (p, 0, 0)),
        pl.BlockSpec((G, EPG, 1), lambda p: (p, 0, 0)),
        pl.BlockSpec((G, 1, N), lambda p: (p, 0, 0)),
        pl.BlockSpec((G, 1, 1), lambda p: (p, 0, 0)),
        _full_spec((STEPS, HID, IN_C)),
        _full_spec((STEPS, HID, 1)),
        _full_spec((STEPS, HID, HID)),
        _full_spec((STEPS, HID, 1)),
        _full_spec((STEPS, IN_C, HID)),
        _full_spec((STEPS, IN_C, 1)),
        _full_spec((64, IN_C)),
        _full_spec((64, 1)),
        _full_spec((32, 64)),
        _full_spec((32, 1)),
        _full_spec((16, 32)),
        _full_spec((16, 1)),
        _full_spec((1, 16)),
        _full_spec((1, 1)),
    ]
    out = pl.pallas_call(
        _main_kernel,
        grid=grid,
        in_specs=in_specs,
        out_specs=pl.BlockSpec((1, 1, G), lambda p: (p, 0, 0)),
        out_shape=jax.ShapeDtypeStruct((B // G, 1, G), jnp.float32),
        compiler_params=pltpu.CompilerParams(
            dimension_semantics=("arbitrary",),
        ),
    )(cnt, xT, srow, drow, scol, dcol, fdx, natsr,
      lin_bf, linb, g1_bf, g1b, g2_bf, g2b,
      flT, flb, m1T, m1b, m2T, m2b, m3T, m3b)
    return out.reshape(B, 1)


# speedup vs baseline: 3.0938x; 1.0673x over previous
import jax
import jax.numpy as jnp
from jax.experimental import pallas as pl
from jax.experimental.pallas import tpu as pltpu

B = 256
N = 256
EPG = 512
IN_C = 5
HID = 512
STEPS = 4
G = 8


def _count_kernel(src_ref, dst_ref, out_ref):
    eq = (src_ref[...] == dst_ref[...]).astype(jnp.float32)
    t = jnp.sum(eq, axis=1, keepdims=True)
    out_ref[...] = jnp.sum(t, axis=0, keepdims=True)


def _main_kernel(cnt_ref, xT_ref, srow_ref, drow_ref, scol_ref, dcol_ref,
                 fdx_ref, nats_ref,
                 lin_bf_ref, linb_ref, g1_bf_ref, g1b_ref, g2_bf_ref, g2b_ref,
                 flT_ref, flb_ref, m1T_ref, m1b_ref, m2T_ref, m2b_ref,
                 m3T_ref, m3b_ref, out_ref):
    p = pl.program_id(0)
    x = jnp.concatenate([xT_ref[g] for g in range(G)], axis=1)

    n_iota_r = jax.lax.broadcasted_iota(jnp.int32, (N, EPG), 0)
    n_iota_c = jax.lax.broadcasted_iota(jnp.int32, (EPG, N), 1)
    ir = jax.lax.broadcasted_iota(jnp.int32, (N, N), 0)
    ic = jax.lax.broadcasted_iota(jnp.int32, (N, N), 1)
    eye = jnp.where(ir == ic, 1.0, 0.0)
    STs, Ds = [], []
    for g in range(G):
        srow = srow_ref[g]
        drow = drow_ref[g]
        scol = scol_ref[g]
        dcol = dcol_ref[g]
        ST = jnp.where((n_iota_r == srow) & (srow != drow), 1.0, 0.0)
        D = jnp.where((n_iota_c == dcol) & (scol != dcol), 1.0, 0.0)
        c_extra = jnp.where((p == 0) & (g == 0), cnt_ref[...], 0.0)
        eye_d = eye + jnp.where((ir == 0) & (ic == 0), c_extra, 0.0)
        STs.append(jnp.concatenate([ST, eye], axis=1))
        Ds.append(jnp.concatenate([D, eye_d], axis=0))

    def prop(v):
        outs = []
        for g in range(G):
            vg = v[:, g * N:(g + 1) * N]
            tmp = jnp.dot(vg, STs[g], preferred_element_type=jnp.float32,
                    precision=jax.lax.Precision.HIGHEST)
            outs.append(jnp.dot(tmp, Ds[g],
                    preferred_element_type=jnp.float32,
                    precision=jax.lax.Precision.HIGHEST))
        return jnp.concatenate(outs, axis=1)

    for gc in range(STEPS):
        if gc > 0:
            h = jnp.tanh(jnp.dot(lin_bf_ref[gc], x.astype(jnp.bfloat16),
                                 preferred_element_type=jnp.float32)
                         + linb_ref[gc])
            h = jnp.tanh(jnp.dot(g1_bf_ref[gc], h.astype(jnp.bfloat16),
                                 preferred_element_type=jnp.float32)
                         + g1b_ref[gc])
            x = jnp.dot(g2_bf_ref[gc], h.astype(jnp.bfloat16),
                        preferred_element_type=jnp.float32) + g2b_ref[gc]
        x = prop(x)

    lane = jax.lax.broadcasted_iota(jnp.int32, (1, N), 1)
    ms = []
    for g in range(G):
        xg = x[:, g * N:(g + 1) * N]
        mask = (lane < nats_ref[g]) & (fdx_ref[g] == 1)
        m = jnp.max(jnp.where(mask, xg, -jnp.inf), axis=1, keepdims=True)
        ms.append(jnp.where(jnp.isfinite(m), m, 0.0))
    m = jnp.concatenate(ms, axis=1)

    h = jnp.tanh(jnp.dot(flT_ref[...], m.astype(jnp.bfloat16),
                         preferred_element_type=jnp.float32) + flb_ref[...])
    h = jnp.tanh(jnp.dot(m1T_ref[...], h.astype(jnp.bfloat16),
                         preferred_element_type=jnp.float32) + m1b_ref[...])
    h = jnp.tanh(jnp.dot(m2T_ref[...], h.astype(jnp.bfloat16),
                         preferred_element_type=jnp.float32) + m2b_ref[...])
    o = jnp.dot(m3T_ref[...], h.astype(jnp.bfloat16),
                preferred_element_type=jnp.float32) + m3b_ref[...]
    out_ref[0] = o


def _full_spec(shape):
    nd = len(shape)
    return pl.BlockSpec(shape, lambda p, _nd=nd: (0,) * _nd)


def kernel(inputs, labels, rval, findex, nats, lin_W, lin_b, g1_W, g1_b,
           g2_W, g2_b, fl_W, fl_b, m1_W, m1_b, m2_W, m2_b, m3_W, m3_b):
    src = labels[:, :, 0]
    dst = labels[:, :, 1]
    srow = src.reshape(B, 1, EPG)
    drow = dst.reshape(B, 1, EPG)
    scol = src.reshape(B, EPG, 1)
    dcol = dst.reshape(B, EPG, 1)
    xT = inputs.transpose(0, 2, 1)
    fdx = findex[:, :, 0].reshape(B, 1, N)
    natsr = nats.reshape(B, 1, 1)

    lin_bf = lin_W.transpose(0, 2, 1).astype(jnp.bfloat16)
    linb = lin_b[:, :, None]
    g1_bf = g1_W.transpose(0, 2, 1).astype(jnp.bfloat16)
    g1b = g1_b[:, :, None]
    g2_bf = g2_W.transpose(0, 2, 1).astype(jnp.bfloat16)
    g2b = g2_b[:, :, None]
    flT = fl_W.T.astype(jnp.bfloat16)
    flb = fl_b[:, None]
    m1T = m1_W.T.astype(jnp.bfloat16)
    m1b = m1_b[:, None]
    m2T = m2_W.T.astype(jnp.bfloat16)
    m2b = m2_b[:, None]
    m3T = m3_W.T.astype(jnp.bfloat16)
    m3b = m3_b[:, None]

    cnt = pl.pallas_call(
        _count_kernel,
        out_shape=jax.ShapeDtypeStruct((1, 1), jnp.float32),
    )(src, dst)

    grid = (B // G,)
    in_specs = [
        _full_spec((1, 1)),
        pl.BlockSpec((G, IN_C, N), lambda p: (p, 0, 0)),
        pl.BlockSpec((G, 1, EPG), lambda p: (p, 0, 0)),
        pl.BlockSpec((G, 1, EPG), lambda p: (p, 0, 0)),
        pl.BlockSpec((G, EPG, 1), lambda p: (p, 0, 0)),
        pl.BlockSpec((G, EPG, 1), lambda p: (p, 0, 0)),
        pl.BlockSpec((G, 1, N), lambda p: (p, 0, 0)),
        pl.BlockSpec((G, 1, 1), lambda p: (p, 0, 0)),
        _full_spec((STEPS, HID, IN_C)),
        _full_spec((STEPS, HID, 1)),
        _full_spec((STEPS, HID, HID)),
        _full_spec((STEPS, HID, 1)),
        _full_spec((STEPS, IN_C, HID)),
        _full_spec((STEPS, IN_C, 1)),
        _full_spec((64, IN_C)),
        _full_spec((64, 1)),
        _full_spec((32, 64)),
        _full_spec((32, 1)),
        _full_spec((16, 32)),
        _full_spec((16, 1)),
        _full_spec((1, 16)),
        _full_spec((1, 1)),
    ]
    out = pl.pallas_call(
        _main_kernel,
        grid=grid,
        in_specs=in_specs,
        out_specs=pl.BlockSpec((1, 1, G), lambda p: (p, 0, 0)),
        out_shape=jax.ShapeDtypeStruct((B // G, 1, G), jnp.float32),
        compiler_params=pltpu.CompilerParams(
            dimension_semantics=("arbitrary",),
        ),
    )(cnt, xT, srow, drow, scol, dcol, fdx, natsr,
      lin_bf, linb, g1_bf, g1b, g2_bf, g2b,
      flT, flb, m1T, m1b, m2T, m2b, m3T, m3b)
    return out.reshape(B, 1)
